# Initial kernel scaffold; baseline (speedup 1.0000x reference)
import jax
import jax.numpy as jnp
from jax.experimental import pallas as pl


def kernel(x, edge_index, batch, W1, b1, W2, b2, W3, b3, Wl1, bl1, Wl2, bl2, Wl3, bl3):
    raise NotImplementedError("write your pallas kernel here")


# speedup vs baseline: 1.0379x; 1.0379x over previous
import functools

import jax
import jax.numpy as jnp
from jax.experimental import pallas as pl
from jax.experimental.pallas import tpu as pltpu

N = 10000
E = 320000
D_IN = 128
HIDDEN = 256
NUM_GRAPHS = 64

POOL_CHUNK = 1000


def _pool_mlp_body(batch_ref, h1_ref, h2_ref, h3_ref,
                   Wl1_ref, bl1_ref, Wl2_ref, bl2_ref, Wl3_ref, bl3_ref,
                   out_ref, acc_ref, cnt_ref):
    i = pl.program_id(0)

    @pl.when(i == 0)
    def _init():
        acc_ref[...] = jnp.zeros_like(acc_ref)
        cnt_ref[...] = jnp.zeros_like(cnt_ref)

    b = batch_ref[0]
    onehot = (b[0][None, :] == jax.lax.broadcasted_iota(
        jnp.int32, (NUM_GRAPHS, POOL_CHUNK), 0)).astype(jnp.float32)
    hcat = jnp.concatenate([h1_ref[...], h2_ref[...], h3_ref[...]], axis=-1)
    acc_ref[...] += jax.lax.dot(onehot, hcat,
                                preferred_element_type=jnp.float32)
    cnt_ref[...] += jnp.sum(onehot, axis=1, keepdims=True)

    @pl.when(i == pl.num_programs(0) - 1)
    def _finish():
        pooled = acc_ref[...] / jnp.maximum(cnt_ref[...], 1.0)
        o = jnp.maximum(
            jax.lax.dot(pooled, Wl1_ref[...],
                        preferred_element_type=jnp.float32) + bl1_ref[...], 0.0)
        o = jnp.maximum(
            jax.lax.dot(o, Wl2_ref[...],
                        preferred_element_type=jnp.float32) + bl2_ref[...], 0.0)
        o = jax.lax.dot(o, Wl3_ref[...],
                        preferred_element_type=jnp.float32) + bl3_ref[...]
        m = jnp.max(o, axis=-1, keepdims=True)
        lse = jnp.log(jnp.sum(jnp.exp(o - m), axis=-1, keepdims=True)) + m
        out_ref[...] = o - lse


def _pool_mlp(batch, h1, h2, h3, Wl1, bl1, Wl2, bl2, Wl3, bl3):
    nsteps = N // POOL_CHUNK
    batch3 = batch.reshape(nsteps, 1, POOL_CHUNK)
    return pl.pallas_call(
        _pool_mlp_body,
        grid=(nsteps,),
        in_specs=[
            pl.BlockSpec((1, 1, POOL_CHUNK), lambda i: (i, 0, 0)),
            pl.BlockSpec((POOL_CHUNK, HIDDEN), lambda i: (i, 0)),
            pl.BlockSpec((POOL_CHUNK, HIDDEN), lambda i: (i, 0)),
            pl.BlockSpec((POOL_CHUNK, HIDDEN), lambda i: (i, 0)),
            pl.BlockSpec((3 * HIDDEN, 2 * HIDDEN), lambda i: (0, 0)),
            pl.BlockSpec((1, 2 * HIDDEN), lambda i: (0, 0)),
            pl.BlockSpec((2 * HIDDEN, HIDDEN), lambda i: (0, 0)),
            pl.BlockSpec((1, HIDDEN), lambda i: (0, 0)),
            pl.BlockSpec((HIDDEN, 2), lambda i: (0, 0)),
            pl.BlockSpec((1, 2), lambda i: (0, 0)),
        ],
        out_specs=pl.BlockSpec((NUM_GRAPHS, 2), lambda i: (0, 0)),
        out_shape=jax.ShapeDtypeStruct((NUM_GRAPHS, 2), jnp.float32),
        scratch_shapes=[
            pltpu.VMEM((NUM_GRAPHS, 3 * HIDDEN), jnp.float32),
            pltpu.VMEM((NUM_GRAPHS, 1), jnp.float32),
        ],
    )(batch3, h1, h2, h3, Wl1, bl1[None, :], Wl2, bl2[None, :],
      Wl3, bl3[None, :])


def kernel(x, edge_index, batch, W1, b1, W2, b2, W3, b3,
           Wl1, bl1, Wl2, bl2, Wl3, bl3):
    src = edge_index[0].astype(jnp.int32)
    dst = edge_index[1].astype(jnp.int32)
    batch = batch.astype(jnp.int32)

    deg = jax.ops.segment_sum(jnp.ones((E,), jnp.float32), dst, num_segments=N)
    inv_deg = 1.0 / jnp.clip(deg, 1.0, None)

    def conv(h, W, b):
        msgs = jnp.take(h, src, axis=0)
        agg = jax.ops.segment_sum(msgs, dst, num_segments=N)
        return jax.nn.relu((agg * inv_deg[:, None]) @ W + b)

    h1 = conv(x, W1, b1)
    h2 = conv(h1, W2, b2)
    h3 = conv(h2, W3, b3)

    return _pool_mlp(batch, h1, h2, h3, Wl1, bl1, Wl2, bl2, Wl3, bl3)


# speedup vs baseline: 2.0573x; 1.9823x over previous
import functools

import jax
import jax.numpy as jnp
from jax import lax
from jax.experimental import pallas as pl
from jax.experimental.pallas import tpu as pltpu
from jax.experimental.pallas import tpu_sc as plsc

N = 10000
E = 320000
D_IN = 128
HIDDEN = 256
NUM_GRAPHS = 64

NC = 2
NS = 16
NW = NC * NS
NPT = 320
NT = NW * NPT
K = 64
EPAD = 128
EP = E + EPAD

POOL_CHUNK = 1000
ROW_BLK = 400


NSC = NS * NPT


def _sc_agg_body(with_deg, D, *refs):
    if with_deg:
        (h_hbm, srcp_hbm, dstp_hbm, toff_hbm, out_hbm, deg_hbm,
         toff_v, src_v, dst_v, dloc_v, ebuf, acc, deg2, gsem) = refs
    else:
        (h_hbm, srcp_hbm, dstp_hbm, toff_hbm, out_hbm,
         toff_v, src_v, dst_v, dloc_v, ebuf, acc, gsem) = refs
        deg2 = deg_hbm = None

    cid = lax.axis_index("c")
    sid = lax.axis_index("s")
    wid = cid * NS + sid
    n_lo = wid * NPT

    pltpu.sync_copy(toff_hbm, toff_v)
    tv = toff_v[pl.ds(wid, 16)]
    e_lo = tv[0]
    e_hi = tv[1]
    e0 = (e_lo // 8) * 8
    nch = (e_hi - e0 + (K - 1)) // K

    z = jnp.zeros((16,), jnp.float32)
    one = jnp.ones((16,), jnp.float32)

    def zrow(r, c):
        for j in range(D // 16):
            acc[r, pl.ds(j * 16, 16)] = z
        if with_deg:
            deg2[r, :] = z
        return c

    lax.fori_loop(0, NPT + 1, zrow, 0)

    def chunk(ci, c):
        e = pl.multiple_of(e0 + ci * K, 8)
        pltpu.sync_copy(srcp_hbm.at[pl.ds(e, K)], src_v)
        pltpu.sync_copy(dstp_hbm.at[pl.ds(e, K)], dst_v)
        pltpu.async_copy(h_hbm.at[src_v], ebuf, gsem).wait()
        for j in range(K // 16):
            d = dst_v[pl.ds(j * 16, 16)]
            inr = (d >= n_lo) & (d < n_lo + NPT)
            dloc_v[pl.ds(j * 16, 16)] = jnp.where(inr, d - n_lo, NPT)

        def edge(l, c2):
            dl = dloc_v[pl.ds(l, 16)][0]
            for cc in range(D // 16):
                plsc.addupdate(acc.at[dl, pl.ds(cc * 16, 16)],
                               ebuf[l, pl.ds(cc * 16, 16)])
            if with_deg:
                plsc.addupdate(deg2.at[dl, :], one)
            return c2

        lax.fori_loop(0, K, edge, 0)
        return c

    lax.fori_loop(0, nch, chunk, 0)

    pltpu.sync_copy(acc.at[pl.ds(0, NPT)], out_hbm.at[pl.ds(n_lo, NPT)])
    if with_deg:
        pltpu.sync_copy(deg2.at[pl.ds(0, NPT)], deg_hbm.at[pl.ds(n_lo, NPT)])


def _sc_agg(h, src_p, dst_p, toff, D, with_deg):
    mesh = plsc.VectorSubcoreMesh(core_axis_name="c", subcore_axis_name="s")
    out_type = [jax.ShapeDtypeStruct((NT, D), jnp.float32)]
    scratch = [
        pltpu.VMEM((40,), jnp.int32),
        pltpu.VMEM((K,), jnp.int32),
        pltpu.VMEM((K,), jnp.int32),
        pltpu.VMEM((K + 16,), jnp.int32),
        pltpu.VMEM((K, D), jnp.float32),
        pltpu.VMEM((NPT + 1, D), jnp.float32),
    ]
    if with_deg:
        out_type.append(jax.ShapeDtypeStruct((NT, 16), jnp.float32))
        scratch.append(pltpu.VMEM((NPT + 1, 16), jnp.float32))
    scratch.append(pltpu.SemaphoreType.DMA)

    fn = pl.kernel(
        functools.partial(_sc_agg_body, with_deg, D),
        out_type=tuple(out_type),
        mesh=mesh,
        scratch_types=tuple(scratch),
    )
    return fn(h, src_p, dst_p, toff)


def _ml_body(agg_ref, deg_ref, W_ref, b_ref, out_ref):
    d = deg_ref[...][:, 0:1]
    inv = 1.0 / jnp.maximum(d, 1.0)
    out_ref[...] = jnp.maximum(
        lax.dot(agg_ref[...] * inv, W_ref[...],
                preferred_element_type=jnp.float32) + b_ref[...], 0.0)


def _mean_linear(agg, deg2, W, b, D):
    nblk = N // ROW_BLK
    return pl.pallas_call(
        _ml_body,
        grid=(nblk,),
        in_specs=[
            pl.BlockSpec((ROW_BLK, D), lambda i: (i, 0)),
            pl.BlockSpec((ROW_BLK, 16), lambda i: (i, 0)),
            pl.BlockSpec((D, HIDDEN), lambda i: (0, 0)),
            pl.BlockSpec((1, HIDDEN), lambda i: (0, 0)),
        ],
        out_specs=pl.BlockSpec((ROW_BLK, HIDDEN), lambda i: (i, 0)),
        out_shape=jax.ShapeDtypeStruct((N, HIDDEN), jnp.float32),
    )(agg, deg2, W, b[None, :])


def _pool_mlp_body(batch_ref, h1_ref, h2_ref, h3_ref,
                   Wl1_ref, bl1_ref, Wl2_ref, bl2_ref, Wl3_ref, bl3_ref,
                   out_ref, acc_ref, cnt_ref):
    i = pl.program_id(0)

    @pl.when(i == 0)
    def _init():
        acc_ref[...] = jnp.zeros_like(acc_ref)
        cnt_ref[...] = jnp.zeros_like(cnt_ref)

    b = batch_ref[0]
    onehot = (b[0][None, :] == lax.broadcasted_iota(
        jnp.int32, (NUM_GRAPHS, POOL_CHUNK), 0)).astype(jnp.float32)
    hcat = jnp.concatenate([h1_ref[...], h2_ref[...], h3_ref[...]], axis=-1)
    acc_ref[...] += lax.dot(onehot, hcat, preferred_element_type=jnp.float32)
    cnt_ref[...] += jnp.sum(onehot, axis=1, keepdims=True)

    @pl.when(i == pl.num_programs(0) - 1)
    def _finish():
        pooled = acc_ref[...] / jnp.maximum(cnt_ref[...], 1.0)
        o = jnp.maximum(
            lax.dot(pooled, Wl1_ref[...],
                    preferred_element_type=jnp.float32) + bl1_ref[...], 0.0)
        o = jnp.maximum(
            lax.dot(o, Wl2_ref[...],
                    preferred_element_type=jnp.float32) + bl2_ref[...], 0.0)
        o = lax.dot(o, Wl3_ref[...],
                    preferred_element_type=jnp.float32) + bl3_ref[...]
        m = jnp.max(o, axis=-1, keepdims=True)
        lse = jnp.log(jnp.sum(jnp.exp(o - m), axis=-1, keepdims=True)) + m
        out_ref[...] = o - lse


def _pool_mlp(batch, h1, h2, h3, Wl1, bl1, Wl2, bl2, Wl3, bl3):
    nsteps = N // POOL_CHUNK
    batch3 = batch.reshape(nsteps, 1, POOL_CHUNK)
    return pl.pallas_call(
        _pool_mlp_body,
        grid=(nsteps,),
        in_specs=[
            pl.BlockSpec((1, 1, POOL_CHUNK), lambda i: (i, 0, 0)),
            pl.BlockSpec((POOL_CHUNK, HIDDEN), lambda i: (i, 0)),
            pl.BlockSpec((POOL_CHUNK, HIDDEN), lambda i: (i, 0)),
            pl.BlockSpec((POOL_CHUNK, HIDDEN), lambda i: (i, 0)),
            pl.BlockSpec((3 * HIDDEN, 2 * HIDDEN), lambda i: (0, 0)),
            pl.BlockSpec((1, 2 * HIDDEN), lambda i: (0, 0)),
            pl.BlockSpec((2 * HIDDEN, HIDDEN), lambda i: (0, 0)),
            pl.BlockSpec((1, HIDDEN), lambda i: (0, 0)),
            pl.BlockSpec((HIDDEN, 2), lambda i: (0, 0)),
            pl.BlockSpec((1, 2), lambda i: (0, 0)),
        ],
        out_specs=pl.BlockSpec((NUM_GRAPHS, 2), lambda i: (0, 0)),
        out_shape=jax.ShapeDtypeStruct((NUM_GRAPHS, 2), jnp.float32),
        scratch_shapes=[
            pltpu.VMEM((NUM_GRAPHS, 3 * HIDDEN), jnp.float32),
            pltpu.VMEM((NUM_GRAPHS, 1), jnp.float32),
        ],
    )(batch3, h1, h2, h3, Wl1, bl1[None, :], Wl2, bl2[None, :],
      Wl3, bl3[None, :])


def kernel(x, edge_index, batch, W1, b1, W2, b2, W3, b3,
           Wl1, bl1, Wl2, bl2, Wl3, bl3):
    src = edge_index[0].astype(jnp.int32)
    dst = edge_index[1].astype(jnp.int32)
    batch = batch.astype(jnp.int32)

    order = jnp.argsort(dst)
    src_s = jnp.take(src, order)
    dst_s = jnp.take(dst, order)
    src_p = jnp.concatenate([src_s, jnp.zeros((EPAD,), jnp.int32)])
    dst_p = jnp.concatenate([dst_s, jnp.full((EPAD,), N, jnp.int32)])
    bounds = jnp.arange(33, dtype=jnp.int32) * NPT
    toff = jnp.searchsorted(dst_s, bounds).astype(jnp.int32)
    toff = jnp.concatenate([toff, jnp.full((7,), E, jnp.int32)])

    agg1, deg2 = _sc_agg(x, src_p, dst_p, toff, D_IN, True)
    h1 = _mean_linear(agg1, deg2, W1, b1, D_IN)
    (agg2,) = _sc_agg(h1, src_p, dst_p, toff, HIDDEN, False)
    h2 = _mean_linear(agg2, deg2, W2, b2, HIDDEN)
    (agg3,) = _sc_agg(h2, src_p, dst_p, toff, HIDDEN, False)
    h3 = _mean_linear(agg3, deg2, W3, b3, HIDDEN)

    return _pool_mlp(batch, h1, h2, h3, Wl1, bl1, Wl2, bl2, Wl3, bl3)


# speedup vs baseline: 2.6011x; 1.2643x over previous
import functools

import jax
import jax.numpy as jnp
from jax import lax
from jax.experimental import pallas as pl
from jax.experimental.pallas import tpu as pltpu
from jax.experimental.pallas import tpu_sc as plsc

N = 10000
E = 320000
D_IN = 128
HIDDEN = 256
NUM_GRAPHS = 64

NC = 2
NS = 16
NW = NC * NS
NPT = 320
NT = NW * NPT
K = 64
IB = 1024
EPAD = 1536
EP = E + EPAD

POOL_CHUNK = 1000
ROW_BLK = 400


NSC = NS * NPT


def _sc_agg_body(with_deg, D, *refs):
    if with_deg:
        (h_hbm, srcp_hbm, dstp_hbm, toff_hbm, out_hbm, deg_hbm,
         toff_v, src_v, dst_v, dloc_v, ebuf, acc, deg2, gsem) = refs
    else:
        (h_hbm, srcp_hbm, dstp_hbm, toff_hbm, out_hbm,
         toff_v, src_v, dst_v, dloc_v, ebuf, acc, gsem) = refs
        deg2 = deg_hbm = None

    cid = lax.axis_index("c")
    sid = lax.axis_index("s")
    wid = cid * NS + sid
    n_lo = wid * NPT

    pltpu.sync_copy(toff_hbm, toff_v)
    tv = toff_v[pl.ds(wid, 16)]
    e_lo = tv[0]
    e_hi = tv[1]
    e0 = (e_lo // 8) * 8
    nic = (e_hi - e0 + (IB - 1)) // IB

    z = jnp.zeros((16,), jnp.float32)
    one = jnp.ones((16,), jnp.float32)

    def zrow(r, c):
        for j in range(D // 16):
            acc[r, pl.ds(j * 16, 16)] = z
        if with_deg:
            deg2[r, :] = z
        return c

    lax.fori_loop(0, NPT + 1, zrow, 0)

    NCH = IB // K

    def iblock(ic, c):
        eb = pl.multiple_of(e0 + ic * IB, 8)
        pltpu.sync_copy(srcp_hbm.at[pl.ds(eb, IB)], src_v)
        pltpu.sync_copy(dstp_hbm.at[pl.ds(eb, IB)], dst_v)
        pltpu.async_copy(h_hbm.at[src_v.at[pl.ds(0, K)]],
                         ebuf.at[0], gsem.at[0])

        def gchunk(g, c2):
            b = lax.rem(g, 2)
            nb = 1 - b
            pltpu.make_async_copy(h_hbm.at[src_v.at[pl.ds(g * K, K)]],
                                  ebuf.at[b], gsem.at[b]).wait()

            @pl.when(g + 1 < NCH)
            def _issue_next():
                pltpu.async_copy(
                    h_hbm.at[src_v.at[pl.ds((g + 1) * K, K)]],
                    ebuf.at[nb], gsem.at[nb])

            for j in range(K // 16):
                d = dst_v[pl.ds(g * K + j * 16, 16)]
                inr = (d >= n_lo) & (d < n_lo + NPT)
                dvec = jnp.where(inr, d - n_lo, NPT)
                for l in range(16):
                    dl = dvec[l]
                    row = j * 16 + l
                    for cc in range(D // 16):
                        plsc.addupdate(acc.at[dl, pl.ds(cc * 16, 16)],
                                       ebuf[b, row, pl.ds(cc * 16, 16)])
                    if with_deg:
                        plsc.addupdate(deg2.at[dl, :], one)
            return c2

        lax.fori_loop(0, NCH, gchunk, 0)
        return c

    lax.fori_loop(0, nic, iblock, 0)

    pltpu.sync_copy(acc.at[pl.ds(0, NPT)], out_hbm.at[pl.ds(n_lo, NPT)])
    if with_deg:
        pltpu.sync_copy(deg2.at[pl.ds(0, NPT)], deg_hbm.at[pl.ds(n_lo, NPT)])


def _sc_agg(h, src_p, dst_p, toff, D, with_deg):
    mesh = plsc.VectorSubcoreMesh(core_axis_name="c", subcore_axis_name="s")
    out_type = [jax.ShapeDtypeStruct((NT, D), jnp.float32)]
    scratch = [
        pltpu.VMEM((40,), jnp.int32),
        pltpu.VMEM((IB,), jnp.int32),
        pltpu.VMEM((IB,), jnp.int32),
        pltpu.VMEM((16,), jnp.int32),
        pltpu.VMEM((2, K, D), jnp.float32),
        pltpu.VMEM((NPT + 1, D), jnp.float32),
    ]
    if with_deg:
        out_type.append(jax.ShapeDtypeStruct((NT, 16), jnp.float32))
        scratch.append(pltpu.VMEM((NPT + 1, 16), jnp.float32))
    scratch.append(pltpu.SemaphoreType.DMA((2,)))

    fn = pl.kernel(
        functools.partial(_sc_agg_body, with_deg, D),
        out_type=tuple(out_type),
        mesh=mesh,
        scratch_types=tuple(scratch),
    )
    return fn(h, src_p, dst_p, toff)


def _ml_body(agg_ref, deg_ref, W_ref, b_ref, out_ref):
    d = deg_ref[...][:, 0:1]
    inv = 1.0 / jnp.maximum(d, 1.0)
    out_ref[...] = jnp.maximum(
        lax.dot(agg_ref[...] * inv, W_ref[...],
                preferred_element_type=jnp.float32) + b_ref[...], 0.0)


def _mean_linear(agg, deg2, W, b, D):
    nblk = N // ROW_BLK
    return pl.pallas_call(
        _ml_body,
        grid=(nblk,),
        in_specs=[
            pl.BlockSpec((ROW_BLK, D), lambda i: (i, 0)),
            pl.BlockSpec((ROW_BLK, 16), lambda i: (i, 0)),
            pl.BlockSpec((D, HIDDEN), lambda i: (0, 0)),
            pl.BlockSpec((1, HIDDEN), lambda i: (0, 0)),
        ],
        out_specs=pl.BlockSpec((ROW_BLK, HIDDEN), lambda i: (i, 0)),
        out_shape=jax.ShapeDtypeStruct((N, HIDDEN), jnp.float32),
    )(agg, deg2, W, b[None, :])


def _pool_mlp_body(batch_ref, h1_ref, h2_ref, h3_ref,
                   Wl1_ref, bl1_ref, Wl2_ref, bl2_ref, Wl3_ref, bl3_ref,
                   out_ref, acc_ref, cnt_ref):
    i = pl.program_id(0)

    @pl.when(i == 0)
    def _init():
        acc_ref[...] = jnp.zeros_like(acc_ref)
        cnt_ref[...] = jnp.zeros_like(cnt_ref)

    b = batch_ref[0]
    onehot = (b[0][None, :] == lax.broadcasted_iota(
        jnp.int32, (NUM_GRAPHS, POOL_CHUNK), 0)).astype(jnp.float32)
    hcat = jnp.concatenate([h1_ref[...], h2_ref[...], h3_ref[...]], axis=-1)
    acc_ref[...] += lax.dot(onehot, hcat, preferred_element_type=jnp.float32)
    cnt_ref[...] += jnp.sum(onehot, axis=1, keepdims=True)

    @pl.when(i == pl.num_programs(0) - 1)
    def _finish():
        pooled = acc_ref[...] / jnp.maximum(cnt_ref[...], 1.0)
        o = jnp.maximum(
            lax.dot(pooled, Wl1_ref[...],
                    preferred_element_type=jnp.float32) + bl1_ref[...], 0.0)
        o = jnp.maximum(
            lax.dot(o, Wl2_ref[...],
                    preferred_element_type=jnp.float32) + bl2_ref[...], 0.0)
        o = lax.dot(o, Wl3_ref[...],
                    preferred_element_type=jnp.float32) + bl3_ref[...]
        m = jnp.max(o, axis=-1, keepdims=True)
        lse = jnp.log(jnp.sum(jnp.exp(o - m), axis=-1, keepdims=True)) + m
        out_ref[...] = o - lse


def _pool_mlp(batch, h1, h2, h3, Wl1, bl1, Wl2, bl2, Wl3, bl3):
    nsteps = N // POOL_CHUNK
    batch3 = batch.reshape(nsteps, 1, POOL_CHUNK)
    return pl.pallas_call(
        _pool_mlp_body,
        grid=(nsteps,),
        in_specs=[
            pl.BlockSpec((1, 1, POOL_CHUNK), lambda i: (i, 0, 0)),
            pl.BlockSpec((POOL_CHUNK, HIDDEN), lambda i: (i, 0)),
            pl.BlockSpec((POOL_CHUNK, HIDDEN), lambda i: (i, 0)),
            pl.BlockSpec((POOL_CHUNK, HIDDEN), lambda i: (i, 0)),
            pl.BlockSpec((3 * HIDDEN, 2 * HIDDEN), lambda i: (0, 0)),
            pl.BlockSpec((1, 2 * HIDDEN), lambda i: (0, 0)),
            pl.BlockSpec((2 * HIDDEN, HIDDEN), lambda i: (0, 0)),
            pl.BlockSpec((1, HIDDEN), lambda i: (0, 0)),
            pl.BlockSpec((HIDDEN, 2), lambda i: (0, 0)),
            pl.BlockSpec((1, 2), lambda i: (0, 0)),
        ],
        out_specs=pl.BlockSpec((NUM_GRAPHS, 2), lambda i: (0, 0)),
        out_shape=jax.ShapeDtypeStruct((NUM_GRAPHS, 2), jnp.float32),
        scratch_shapes=[
            pltpu.VMEM((NUM_GRAPHS, 3 * HIDDEN), jnp.float32),
            pltpu.VMEM((NUM_GRAPHS, 1), jnp.float32),
        ],
    )(batch3, h1, h2, h3, Wl1, bl1[None, :], Wl2, bl2[None, :],
      Wl3, bl3[None, :])


def kernel(x, edge_index, batch, W1, b1, W2, b2, W3, b3,
           Wl1, bl1, Wl2, bl2, Wl3, bl3):
    src = edge_index[0].astype(jnp.int32)
    dst = edge_index[1].astype(jnp.int32)
    batch = batch.astype(jnp.int32)

    order = jnp.argsort(dst)
    src_s = jnp.take(src, order)
    dst_s = jnp.take(dst, order)
    src_p = jnp.concatenate([src_s, jnp.zeros((EPAD,), jnp.int32)])
    dst_p = jnp.concatenate([dst_s, jnp.full((EPAD,), N, jnp.int32)])
    bounds = jnp.arange(33, dtype=jnp.int32) * NPT
    toff = jnp.searchsorted(dst_s, bounds).astype(jnp.int32)
    toff = jnp.concatenate([toff, jnp.full((7,), E, jnp.int32)])

    agg1, deg2 = _sc_agg(x, src_p, dst_p, toff, D_IN, True)
    h1 = _mean_linear(agg1, deg2, W1, b1, D_IN)
    (agg2,) = _sc_agg(h1, src_p, dst_p, toff, HIDDEN, False)
    h2 = _mean_linear(agg2, deg2, W2, b2, HIDDEN)
    (agg3,) = _sc_agg(h2, src_p, dst_p, toff, HIDDEN, False)
    h3 = _mean_linear(agg3, deg2, W3, b3, HIDDEN)

    return _pool_mlp(batch, h1, h2, h3, Wl1, bl1, Wl2, bl2, Wl3, bl3)


# speedup vs baseline: 3.4385x; 1.3219x over previous
import functools

import jax
import jax.numpy as jnp
from jax import lax
from jax.experimental import pallas as pl
from jax.experimental.pallas import tpu as pltpu
from jax.experimental.pallas import tpu_sc as plsc

N = 10000
E = 320000
D_IN = 128
HIDDEN = 256
NUM_GRAPHS = 64

NC = 2
NS = 16
NW = NC * NS
NPT = 320
NT = NW * NPT
K = 64
IB = 1024
EPAD = 1536
EP = E + EPAD

POOL_CHUNK = 1000
ROW_BLK = 400


NSC = NS * NPT


def _sc_agg_body(with_deg, D, *refs):
    if with_deg:
        (h_hbm, srcp_hbm, dstp_hbm, toff_hbm, out_hbm, deg_hbm,
         toff_v, src_v, dst_v, ebuf, acc, deg2, gsem) = refs
    else:
        (h_hbm, srcp_hbm, dstp_hbm, toff_hbm, out_hbm,
         toff_v, src_v, dst_v, ebuf, acc, gsem) = refs
        deg_hbm = deg2 = None

    cid = lax.axis_index("c")
    sid = lax.axis_index("s")
    wid = cid * NS + sid
    n_lo = wid * NPT

    pltpu.sync_copy(toff_hbm, toff_v)
    tv = toff_v[pl.ds(wid, 16)]
    e_lo = tv[0]
    e_hi = tv[1]
    e0 = (e_lo // 8) * 8
    nic = (e_hi - e0 + (IB - 1)) // IB

    z = jnp.zeros((16,), jnp.float32)
    one = jnp.ones((16,), jnp.float32)
    NCC = D // 16

    def zrow(r, c):
        for j in range(NCC):
            acc[pl.ds(r * D + j * 16, 16)] = z
        if with_deg:
            deg2[pl.ds(r * 16, 16)] = z
        return c

    lax.fori_loop(0, NPT + 1, zrow, 0)

    NCH = IB // K

    init = (jnp.int32(NPT), (z,) * NCC, z)

    def iblock(ic, carry):
        eb = pl.multiple_of(e0 + ic * IB, 8)
        pltpu.sync_copy(srcp_hbm.at[pl.ds(eb, IB)], src_v)
        pltpu.sync_copy(dstp_hbm.at[pl.ds(eb, IB)], dst_v)
        pltpu.async_copy(h_hbm.at[src_v.at[pl.ds(0, K)]],
                         ebuf.at[0], gsem.at[0])

        def gchunk(g, carry2):
            b = lax.rem(g, 2)
            nb = 1 - b
            pltpu.make_async_copy(h_hbm.at[src_v.at[pl.ds(g * K, K)]],
                                  ebuf.at[b], gsem.at[b]).wait()

            @pl.when(g + 1 < NCH)
            def _issue_next():
                pltpu.async_copy(
                    h_hbm.at[src_v.at[pl.ds((g + 1) * K, K)]],
                    ebuf.at[nb], gsem.at[nb])

            cur_dl, cur, cdeg = carry2
            for j in range(K // 16):
                d = dst_v[pl.ds(g * K + j * 16, 16)]
                inr = (d >= n_lo) & (d < n_lo + NPT)
                dvec = jnp.where(inr, d - n_lo, NPT)
                for l in range(16):
                    dl = dvec[l]
                    row = j * 16 + l
                    diff = dl != cur_dl

                    @pl.when(diff)
                    def _flush(fdl=cur_dl, fcur=cur, fdeg=cdeg):
                        for cc in range(NCC):
                            plsc.addupdate(
                                acc.at[pl.ds(fdl * D + cc * 16, 16)],
                                fcur[cc])
                        if with_deg:
                            plsc.addupdate(deg2.at[pl.ds(fdl * 16, 16)],
                                           fdeg)

                    kf = jnp.where(diff, 0.0, 1.0)
                    cur = tuple(cur[cc] * kf
                                + ebuf[b, row, pl.ds(cc * 16, 16)]
                                for cc in range(NCC))
                    if with_deg:
                        cdeg = cdeg * kf + one
                    cur_dl = dl
            return (cur_dl, cur, cdeg)

        return lax.fori_loop(0, NCH, gchunk, carry)

    cur_dl, cur, cdeg = lax.fori_loop(0, nic, iblock, init)
    for cc in range(NCC):
        plsc.addupdate(acc.at[pl.ds(cur_dl * D + cc * 16, 16)], cur[cc])
    if with_deg:
        plsc.addupdate(deg2.at[pl.ds(cur_dl * 16, 16)], cdeg)

    pltpu.sync_copy(acc.at[pl.ds(0, NPT * D)],
                    out_hbm.at[pl.ds(n_lo * D, NPT * D)])
    if with_deg:
        pltpu.sync_copy(deg2.at[pl.ds(0, NPT * 16)],
                        deg_hbm.at[pl.ds(n_lo * 16, NPT * 16)])


def _sc_agg(h, src_p, dst_p, toff, D, with_deg):
    mesh = plsc.VectorSubcoreMesh(core_axis_name="c", subcore_axis_name="s")
    out_type = [jax.ShapeDtypeStruct((NT * D,), jnp.float32)]
    scratch = [
        pltpu.VMEM((40,), jnp.int32),
        pltpu.VMEM((IB,), jnp.int32),
        pltpu.VMEM((IB,), jnp.int32),
        pltpu.VMEM((2, K, D), jnp.float32),
        pltpu.VMEM(((NPT + 1) * D,), jnp.float32),
    ]
    if with_deg:
        out_type.append(jax.ShapeDtypeStruct((NT * 16,), jnp.float32))
        scratch.append(pltpu.VMEM(((NPT + 1) * 16,), jnp.float32))
    scratch.append(pltpu.SemaphoreType.DMA((2,)))

    fn = pl.kernel(
        functools.partial(_sc_agg_body, with_deg, D),
        out_type=tuple(out_type),
        mesh=mesh,
        scratch_types=tuple(scratch),
    )
    return fn(h, src_p, dst_p, toff)


def _ml_body(agg_ref, deg_ref, W_ref, b_ref, out_ref):
    d = deg_ref[...][:, 0:1]
    inv = 1.0 / jnp.maximum(d, 1.0)
    out_ref[...] = jnp.maximum(
        lax.dot(agg_ref[...] * inv, W_ref[...],
                preferred_element_type=jnp.float32) + b_ref[...], 0.0)


def _mean_linear(agg, deg2, W, b, D):
    nblk = N // ROW_BLK
    return pl.pallas_call(
        _ml_body,
        grid=(nblk,),
        in_specs=[
            pl.BlockSpec((ROW_BLK, D), lambda i: (i, 0)),
            pl.BlockSpec((ROW_BLK, 16), lambda i: (i, 0)),
            pl.BlockSpec((D, HIDDEN), lambda i: (0, 0)),
            pl.BlockSpec((1, HIDDEN), lambda i: (0, 0)),
        ],
        out_specs=pl.BlockSpec((ROW_BLK, HIDDEN), lambda i: (i, 0)),
        out_shape=jax.ShapeDtypeStruct((N, HIDDEN), jnp.float32),
    )(agg, deg2, W, b[None, :])


def _pool_mlp_body(batch_ref, h1_ref, h2_ref, h3_ref,
                   Wl1_ref, bl1_ref, Wl2_ref, bl2_ref, Wl3_ref, bl3_ref,
                   out_ref, acc_ref, cnt_ref):
    i = pl.program_id(0)

    @pl.when(i == 0)
    def _init():
        acc_ref[...] = jnp.zeros_like(acc_ref)
        cnt_ref[...] = jnp.zeros_like(cnt_ref)

    b = batch_ref[0]
    onehot = (b[0][None, :] == lax.broadcasted_iota(
        jnp.int32, (NUM_GRAPHS, POOL_CHUNK), 0)).astype(jnp.float32)
    hcat = jnp.concatenate([h1_ref[...], h2_ref[...], h3_ref[...]], axis=-1)
    acc_ref[...] += lax.dot(onehot, hcat, preferred_element_type=jnp.float32)
    cnt_ref[...] += jnp.sum(onehot, axis=1, keepdims=True)

    @pl.when(i == pl.num_programs(0) - 1)
    def _finish():
        pooled = acc_ref[...] / jnp.maximum(cnt_ref[...], 1.0)
        o = jnp.maximum(
            lax.dot(pooled, Wl1_ref[...],
                    preferred_element_type=jnp.float32) + bl1_ref[...], 0.0)
        o = jnp.maximum(
            lax.dot(o, Wl2_ref[...],
                    preferred_element_type=jnp.float32) + bl2_ref[...], 0.0)
        o = lax.dot(o, Wl3_ref[...],
                    preferred_element_type=jnp.float32) + bl3_ref[...]
        m = jnp.max(o, axis=-1, keepdims=True)
        lse = jnp.log(jnp.sum(jnp.exp(o - m), axis=-1, keepdims=True)) + m
        out_ref[...] = o - lse


def _pool_mlp(batch, h1, h2, h3, Wl1, bl1, Wl2, bl2, Wl3, bl3):
    nsteps = N // POOL_CHUNK
    batch3 = batch.reshape(nsteps, 1, POOL_CHUNK)
    return pl.pallas_call(
        _pool_mlp_body,
        grid=(nsteps,),
        in_specs=[
            pl.BlockSpec((1, 1, POOL_CHUNK), lambda i: (i, 0, 0)),
            pl.BlockSpec((POOL_CHUNK, HIDDEN), lambda i: (i, 0)),
            pl.BlockSpec((POOL_CHUNK, HIDDEN), lambda i: (i, 0)),
            pl.BlockSpec((POOL_CHUNK, HIDDEN), lambda i: (i, 0)),
            pl.BlockSpec((3 * HIDDEN, 2 * HIDDEN), lambda i: (0, 0)),
            pl.BlockSpec((1, 2 * HIDDEN), lambda i: (0, 0)),
            pl.BlockSpec((2 * HIDDEN, HIDDEN), lambda i: (0, 0)),
            pl.BlockSpec((1, HIDDEN), lambda i: (0, 0)),
            pl.BlockSpec((HIDDEN, 2), lambda i: (0, 0)),
            pl.BlockSpec((1, 2), lambda i: (0, 0)),
        ],
        out_specs=pl.BlockSpec((NUM_GRAPHS, 2), lambda i: (0, 0)),
        out_shape=jax.ShapeDtypeStruct((NUM_GRAPHS, 2), jnp.float32),
        scratch_shapes=[
            pltpu.VMEM((NUM_GRAPHS, 3 * HIDDEN), jnp.float32),
            pltpu.VMEM((NUM_GRAPHS, 1), jnp.float32),
        ],
    )(batch3, h1, h2, h3, Wl1, bl1[None, :], Wl2, bl2[None, :],
      Wl3, bl3[None, :])


def kernel(x, edge_index, batch, W1, b1, W2, b2, W3, b3,
           Wl1, bl1, Wl2, bl2, Wl3, bl3):
    src = edge_index[0].astype(jnp.int32)
    dst = edge_index[1].astype(jnp.int32)
    batch = batch.astype(jnp.int32)

    order = jnp.argsort(dst)
    src_s = jnp.take(src, order)
    dst_s = jnp.take(dst, order)
    src_p = jnp.concatenate([src_s, jnp.zeros((EPAD,), jnp.int32)])
    dst_p = jnp.concatenate([dst_s, jnp.full((EPAD,), N, jnp.int32)])
    bounds = jnp.arange(33, dtype=jnp.int32) * NPT
    toff = jnp.searchsorted(dst_s, bounds).astype(jnp.int32)
    toff = jnp.concatenate([toff, jnp.full((7,), E, jnp.int32)])

    agg1, deg2 = _sc_agg(x, src_p, dst_p, toff, D_IN, True)
    agg1 = agg1.reshape(NT, D_IN)
    deg2 = deg2.reshape(NT, 16)
    h1 = _mean_linear(agg1, deg2, W1, b1, D_IN)
    (agg2,) = _sc_agg(h1, src_p, dst_p, toff, HIDDEN, False)
    h2 = _mean_linear(agg2.reshape(NT, HIDDEN), deg2, W2, b2, HIDDEN)
    (agg3,) = _sc_agg(h2, src_p, dst_p, toff, HIDDEN, False)
    h3 = _mean_linear(agg3.reshape(NT, HIDDEN), deg2, W3, b3, HIDDEN)

    return _pool_mlp(batch, h1, h2, h3, Wl1, bl1, Wl2, bl2, Wl3, bl3)


# speedup vs baseline: 3.5478x; 1.0318x over previous
import functools

import jax
import jax.numpy as jnp
from jax import lax
from jax.experimental import pallas as pl
from jax.experimental.pallas import tpu as pltpu
from jax.experimental.pallas import tpu_sc as plsc

N = 10000
E = 320000
D_IN = 128
HIDDEN = 256
NUM_GRAPHS = 64

NC = 2
NS = 16
NW = NC * NS
NPT = 320
NT = NW * NPT
K = 64
IB = 1024
EPAD = 1536
EP = E + EPAD

POOL_CHUNK = 1000
ROW_BLK = 400


NSC = NS * NPT


def _sc_agg_body(with_deg, D, *refs):
    if with_deg:
        (h_hbm, pk_hbm, toff_hbm, out_hbm, deg_hbm,
         toff_v, src_v, pk_v, ebuf, acc, deg2, gsem) = refs
    else:
        (h_hbm, pk_hbm, toff_hbm, out_hbm,
         toff_v, src_v, pk_v, ebuf, acc, gsem) = refs
        deg_hbm = deg2 = None

    cid = lax.axis_index("c")
    sid = lax.axis_index("s")
    wid = cid * NS + sid
    n_lo = wid * NPT

    pltpu.sync_copy(toff_hbm, toff_v)
    tv = toff_v[pl.ds(wid, 16)]
    e_lo = tv[0]
    e_hi = tv[1]
    e0 = (e_lo // 8) * 8
    nic = (e_hi - e0 + (IB - 1)) // IB

    z = jnp.zeros((16,), jnp.float32)
    one = jnp.ones((16,), jnp.float32)
    NCC = D // 16

    def zrow(r, c):
        for j in range(NCC):
            acc[pl.ds(r * D + j * 16, 16)] = z
        if with_deg:
            deg2[pl.ds(r * 16, 16)] = z
        return c

    lax.fori_loop(0, NPT + 1, zrow, 0)

    NCH = IB // K

    init = (jnp.int32(NPT), (z,) * NCC, z)

    def iblock(ic, carry):
        eb = pl.multiple_of(e0 + ic * IB, 8)
        pltpu.sync_copy(pk_hbm.at[pl.ds(eb, IB)], pk_v)
        def unpk(jj, c0):
            pk = pk_v[pl.ds(jj * 16, 16)]
            src_v[pl.ds(jj * 16, 16)] = pk & 16383
            return c0

        lax.fori_loop(0, IB // 16, unpk, 0)
        pltpu.async_copy(h_hbm.at[src_v.at[pl.ds(0, K)]],
                         ebuf.at[0], gsem.at[0])

        def gchunk(g, carry2):
            b = lax.rem(g, 2)
            nb = 1 - b
            pltpu.make_async_copy(h_hbm.at[src_v.at[pl.ds(g * K, K)]],
                                  ebuf.at[b], gsem.at[b]).wait()

            @pl.when(g + 1 < NCH)
            def _issue_next():
                pltpu.async_copy(
                    h_hbm.at[src_v.at[pl.ds((g + 1) * K, K)]],
                    ebuf.at[nb], gsem.at[nb])

            cur_dl, cur, cdeg = carry2
            for j in range(K // 16):
                d = pk_v[pl.ds(g * K + j * 16, 16)] >> 14
                inr = (d >= n_lo) & (d < n_lo + NPT)
                dvec = jnp.where(inr, d - n_lo, NPT)
                for l in range(16):
                    dl = dvec[l]
                    row = j * 16 + l
                    diff = dl != cur_dl

                    @pl.when(diff)
                    def _flush(fdl=cur_dl, fcur=cur, fdeg=cdeg):
                        for cc in range(NCC):
                            plsc.addupdate(
                                acc.at[pl.ds(fdl * D + cc * 16, 16)],
                                fcur[cc])
                        if with_deg:
                            plsc.addupdate(deg2.at[pl.ds(fdl * 16, 16)],
                                           fdeg)

                    kf = jnp.where(diff, 0.0, 1.0)
                    cur = tuple(cur[cc] * kf
                                + ebuf[b, row, pl.ds(cc * 16, 16)]
                                for cc in range(NCC))
                    if with_deg:
                        cdeg = cdeg * kf + one
                    cur_dl = dl
            return (cur_dl, cur, cdeg)

        return lax.fori_loop(0, NCH, gchunk, carry)

    cur_dl, cur, cdeg = lax.fori_loop(0, nic, iblock, init)
    for cc in range(NCC):
        plsc.addupdate(acc.at[pl.ds(cur_dl * D + cc * 16, 16)], cur[cc])
    if with_deg:
        plsc.addupdate(deg2.at[pl.ds(cur_dl * 16, 16)], cdeg)

    pltpu.sync_copy(acc.at[pl.ds(0, NPT * D)],
                    out_hbm.at[pl.ds(n_lo * D, NPT * D)])
    if with_deg:
        pltpu.sync_copy(deg2.at[pl.ds(0, NPT * 16)],
                        deg_hbm.at[pl.ds(n_lo * 16, NPT * 16)])


def _sc_agg(h, pk_p, toff, D, with_deg):
    mesh = plsc.VectorSubcoreMesh(core_axis_name="c", subcore_axis_name="s")
    out_type = [jax.ShapeDtypeStruct((NT * D,), jnp.float32)]
    scratch = [
        pltpu.VMEM((40,), jnp.int32),
        pltpu.VMEM((IB,), jnp.int32),
        pltpu.VMEM((IB,), jnp.int32),
        pltpu.VMEM((2, K, D), jnp.float32),
        pltpu.VMEM(((NPT + 1) * D,), jnp.float32),
    ]
    if with_deg:
        out_type.append(jax.ShapeDtypeStruct((NT * 16,), jnp.float32))
        scratch.append(pltpu.VMEM(((NPT + 1) * 16,), jnp.float32))
    scratch.append(pltpu.SemaphoreType.DMA((2,)))

    fn = pl.kernel(
        functools.partial(_sc_agg_body, with_deg, D),
        out_type=tuple(out_type),
        mesh=mesh,
        scratch_types=tuple(scratch),
    )
    return fn(h, pk_p, toff)


def _ml_body(agg_ref, deg_ref, W_ref, b_ref, out_ref):
    d = deg_ref[...][:, 0:1]
    inv = 1.0 / jnp.maximum(d, 1.0)
    out_ref[...] = jnp.maximum(
        lax.dot(agg_ref[...] * inv, W_ref[...],
                preferred_element_type=jnp.float32) + b_ref[...], 0.0)


def _mean_linear(agg, deg2, W, b, D):
    nblk = N // ROW_BLK
    return pl.pallas_call(
        _ml_body,
        grid=(nblk,),
        in_specs=[
            pl.BlockSpec((ROW_BLK, D), lambda i: (i, 0)),
            pl.BlockSpec((ROW_BLK, 16), lambda i: (i, 0)),
            pl.BlockSpec((D, HIDDEN), lambda i: (0, 0)),
            pl.BlockSpec((1, HIDDEN), lambda i: (0, 0)),
        ],
        out_specs=pl.BlockSpec((ROW_BLK, HIDDEN), lambda i: (i, 0)),
        out_shape=jax.ShapeDtypeStruct((N, HIDDEN), jnp.float32),
    )(agg, deg2, W, b[None, :])


def _pool_mlp_body(batch_ref, h1_ref, h2_ref, h3_ref,
                   Wl1_ref, bl1_ref, Wl2_ref, bl2_ref, Wl3_ref, bl3_ref,
                   out_ref, acc_ref, cnt_ref):
    i = pl.program_id(0)

    @pl.when(i == 0)
    def _init():
        acc_ref[...] = jnp.zeros_like(acc_ref)
        cnt_ref[...] = jnp.zeros_like(cnt_ref)

    b = batch_ref[0]
    onehot = (b[0][None, :] == lax.broadcasted_iota(
        jnp.int32, (NUM_GRAPHS, POOL_CHUNK), 0)).astype(jnp.float32)
    hcat = jnp.concatenate([h1_ref[...], h2_ref[...], h3_ref[...]], axis=-1)
    acc_ref[...] += lax.dot(onehot, hcat, preferred_element_type=jnp.float32)
    cnt_ref[...] += jnp.sum(onehot, axis=1, keepdims=True)

    @pl.when(i == pl.num_programs(0) - 1)
    def _finish():
        pooled = acc_ref[...] / jnp.maximum(cnt_ref[...], 1.0)
        o = jnp.maximum(
            lax.dot(pooled, Wl1_ref[...],
                    preferred_element_type=jnp.float32) + bl1_ref[...], 0.0)
        o = jnp.maximum(
            lax.dot(o, Wl2_ref[...],
                    preferred_element_type=jnp.float32) + bl2_ref[...], 0.0)
        o = lax.dot(o, Wl3_ref[...],
                    preferred_element_type=jnp.float32) + bl3_ref[...]
        m = jnp.max(o, axis=-1, keepdims=True)
        lse = jnp.log(jnp.sum(jnp.exp(o - m), axis=-1, keepdims=True)) + m
        out_ref[...] = o - lse


def _pool_mlp(batch, h1, h2, h3, Wl1, bl1, Wl2, bl2, Wl3, bl3):
    nsteps = N // POOL_CHUNK
    batch3 = batch.reshape(nsteps, 1, POOL_CHUNK)
    return pl.pallas_call(
        _pool_mlp_body,
        grid=(nsteps,),
        in_specs=[
            pl.BlockSpec((1, 1, POOL_CHUNK), lambda i: (i, 0, 0)),
            pl.BlockSpec((POOL_CHUNK, HIDDEN), lambda i: (i, 0)),
            pl.BlockSpec((POOL_CHUNK, HIDDEN), lambda i: (i, 0)),
            pl.BlockSpec((POOL_CHUNK, HIDDEN), lambda i: (i, 0)),
            pl.BlockSpec((3 * HIDDEN, 2 * HIDDEN), lambda i: (0, 0)),
            pl.BlockSpec((1, 2 * HIDDEN), lambda i: (0, 0)),
            pl.BlockSpec((2 * HIDDEN, HIDDEN), lambda i: (0, 0)),
            pl.BlockSpec((1, HIDDEN), lambda i: (0, 0)),
            pl.BlockSpec((HIDDEN, 2), lambda i: (0, 0)),
            pl.BlockSpec((1, 2), lambda i: (0, 0)),
        ],
        out_specs=pl.BlockSpec((NUM_GRAPHS, 2), lambda i: (0, 0)),
        out_shape=jax.ShapeDtypeStruct((NUM_GRAPHS, 2), jnp.float32),
        scratch_shapes=[
            pltpu.VMEM((NUM_GRAPHS, 3 * HIDDEN), jnp.float32),
            pltpu.VMEM((NUM_GRAPHS, 1), jnp.float32),
        ],
    )(batch3, h1, h2, h3, Wl1, bl1[None, :], Wl2, bl2[None, :],
      Wl3, bl3[None, :])


def kernel(x, edge_index, batch, W1, b1, W2, b2, W3, b3,
           Wl1, bl1, Wl2, bl2, Wl3, bl3):
    src = edge_index[0].astype(jnp.int32)
    dst = edge_index[1].astype(jnp.int32)
    batch = batch.astype(jnp.int32)

    packed = jnp.sort(dst * 16384 + src)
    pk_p = jnp.concatenate([packed, jnp.full((EPAD,), N * 16384, jnp.int32)])
    bounds = jnp.arange(33, dtype=jnp.int32) * (NPT * 16384)
    toff = jnp.searchsorted(packed, bounds).astype(jnp.int32)
    toff = jnp.concatenate([toff, jnp.full((7,), E, jnp.int32)])

    agg1, deg2 = _sc_agg(x, pk_p, toff, D_IN, True)
    agg1 = agg1.reshape(NT, D_IN)
    deg2 = deg2.reshape(NT, 16)
    h1 = _mean_linear(agg1, deg2, W1, b1, D_IN)
    (agg2,) = _sc_agg(h1, pk_p, toff, HIDDEN, False)
    h2 = _mean_linear(agg2.reshape(NT, HIDDEN), deg2, W2, b2, HIDDEN)
    (agg3,) = _sc_agg(h2, pk_p, toff, HIDDEN, False)
    h3 = _mean_linear(agg3.reshape(NT, HIDDEN), deg2, W3, b3, HIDDEN)

    return _pool_mlp(batch, h1, h2, h3, Wl1, bl1, Wl2, bl2, Wl3, bl3)
